# baseline (speedup 1.0000x reference)
import jax
import jax.numpy as jnp
from jax import lax
from jax.experimental import pallas as pl
from jax.experimental.pallas import tpu as pltpu
from jax.experimental.pallas import tpu_sc as plsc

_B, _L, _H, _D = 8, 4096, 8, 128
_Q = 8
_NC, _NS = 2, 16
_WPB = (_NC * _NS) // _B
_TPW = _Q // _WPB


def _make_body(with_lengths_out):
  def _scatter_body(len_hbm, nl_hbm, src_hbm, cache_ref, *rest):
    if with_lengths_out:
      ul_hbm, len_v, nl_v, ul_v, row, sem_len, sem_row = rest
    else:
      len_v, nl_v, ul_v, row, sem_len, sem_row = rest
    c = lax.axis_index("c")
    s = lax.axis_index("s")
    wid = s * _NC + c

    pltpu.async_copy(len_hbm, len_v.at[pl.ds(0, _B)], sem_len)
    pltpu.async_copy(nl_hbm, nl_v.at[pl.ds(0, _B)], sem_len)
    pltpu.make_async_copy(len_hbm, len_v.at[pl.ds(0, _B)], sem_len).wait()
    pltpu.make_async_copy(nl_hbm, nl_v.at[pl.ds(0, _B)], sem_len).wait()
    vals_l = len_v[...]
    vals_nl = nl_v[...]

    for b in range(_B):
      l_b = vals_l[b]
      nl_b = vals_nl[b]
      owned = wid // _WPB == b
      for t in range(_TPW):
        j = lax.rem(wid, _WPB) * _TPW + t

        @pl.when(jnp.logical_and(owned, j < nl_b))
        def _copy_row(b=b, j=j, l_b=l_b):
          pltpu.async_copy(src_hbm.at[b, j], row, sem_row).wait()
          pltpu.async_copy(row, cache_ref.at[b, l_b + j], sem_row).wait()

    if with_lengths_out:
      @pl.when(jnp.logical_and(c == 0, s == 0))
      def _update_lengths():
        ul_v[...] = vals_l + vals_nl
        pltpu.sync_copy(ul_v.at[pl.ds(0, _B)], ul_hbm)

  return _scatter_body


_scratch = [
    pltpu.VMEM((16,), jnp.int32),
    pltpu.VMEM((16,), jnp.int32),
    pltpu.VMEM((16,), jnp.int32),
    pltpu.VMEM((_H, _D), jnp.float32),
    pltpu.SemaphoreType.DMA,
    pltpu.SemaphoreType.DMA,
]

_mesh = plsc.VectorSubcoreMesh(
    core_axis_name="c", subcore_axis_name="s",
    num_cores=_NC, num_subcores=_NS)

_sc_scatter_k = pl.kernel(
    _make_body(with_lengths_out=True),
    out_type=jax.ShapeDtypeStruct((_B,), jnp.int32),
    mesh=_mesh,
    scratch_types=_scratch,
)

_sc_scatter_v = pl.kernel(
    _make_body(with_lengths_out=False),
    out_type=(),
    mesh=_mesh,
    scratch_types=_scratch,
)


def kernel(keys, values, lengths, new_keys, new_values, new_lengths):
  k_ref = jax.new_ref(keys)
  v_ref = jax.new_ref(values)
  updated_lengths = _sc_scatter_k(lengths, new_lengths, new_keys, k_ref)
  _sc_scatter_v(lengths, new_lengths, new_values, v_ref)
  return jax.freeze(k_ref), jax.freeze(v_ref), updated_lengths


# speedup vs baseline: 1.0456x; 1.0456x over previous
import jax
import jax.numpy as jnp
from jax import lax
from jax.experimental import pallas as pl
from jax.experimental.pallas import tpu as pltpu
from jax.experimental.pallas import tpu_sc as plsc

_B, _L, _H, _D = 8, 4096, 8, 128
_Q = 8
_NS = 16
_WPB = _NS // _B
_TPW = _Q // _WPB


def _scatter_body(len_hbm, nl_hbm, nk_hbm, nv_hbm, k_ref, v_ref, ul_hbm,
                  len_v, nl_v, ul_v, rowk, rowv, sem_len, sem_g, sem_w):
  wid = lax.axis_index("s")
  b_mine = wid // _WPB
  jlane = lax.rem(wid, _WPB)

  def _slot(t):
    return jlane + _WPB * t

  pltpu.async_copy(len_hbm, len_v.at[pl.ds(0, _B)], sem_len)
  pltpu.async_copy(nl_hbm, nl_v.at[pl.ds(0, _B)], sem_len)
  for t in range(_TPW):
    pltpu.async_copy(nk_hbm.at[b_mine, _slot(t)], rowk.at[t], sem_g)
    pltpu.async_copy(nv_hbm.at[b_mine, _slot(t)], rowv.at[t], sem_g)

  pltpu.make_async_copy(len_hbm, len_v.at[pl.ds(0, _B)], sem_len).wait()
  pltpu.make_async_copy(nl_hbm, nl_v.at[pl.ds(0, _B)], sem_len).wait()
  vals_l = len_v[...]
  vals_nl = nl_v[...]

  for t in range(_TPW):
    pltpu.make_async_copy(nk_hbm.at[b_mine, _slot(t)], rowk.at[t],
                          sem_g).wait()
    pltpu.make_async_copy(nv_hbm.at[b_mine, _slot(t)], rowv.at[t],
                          sem_g).wait()

  def _for_active(fn):
    for b in range(_B):
      l_b = vals_l[b]
      nl_b = vals_nl[b]
      owned = b_mine == b
      for t in range(_TPW):
        j = _slot(t)
        pl.when(jnp.logical_and(owned, j < nl_b))(
            lambda b=b, t=t, j=j, l_b=l_b: fn(b, t, j, l_b))

  def _fire(b, t, j, l_b):
    pltpu.async_copy(rowk.at[t], k_ref.at[b, l_b + j], sem_w)
    pltpu.async_copy(rowv.at[t], v_ref.at[b, l_b + j], sem_w)

  def _drain(b, t, j, l_b):
    pltpu.make_async_copy(rowk.at[t], k_ref.at[b, l_b + j], sem_w).wait()
    pltpu.make_async_copy(rowv.at[t], v_ref.at[b, l_b + j], sem_w).wait()

  _for_active(_fire)

  @pl.when(wid == 0)
  def _update_lengths():
    ul_v[...] = vals_l + vals_nl
    pltpu.sync_copy(ul_v.at[pl.ds(0, _B)], ul_hbm)

  _for_active(_drain)


_sc_scatter = pl.kernel(
    _scatter_body,
    out_type=jax.ShapeDtypeStruct((_B,), jnp.int32),
    mesh=plsc.VectorSubcoreMesh(
        core_axis_name="c", subcore_axis_name="s",
        num_cores=1, num_subcores=_NS),
    scratch_types=[
        pltpu.VMEM((16,), jnp.int32),
        pltpu.VMEM((16,), jnp.int32),
        pltpu.VMEM((16,), jnp.int32),
        pltpu.VMEM((_TPW, _H, _D), jnp.float32),
        pltpu.VMEM((_TPW, _H, _D), jnp.float32),
        pltpu.SemaphoreType.DMA,
        pltpu.SemaphoreType.DMA,
        pltpu.SemaphoreType.DMA,
    ],
)


def kernel(keys, values, lengths, new_keys, new_values, new_lengths):
  k_ref = jax.new_ref(keys)
  v_ref = jax.new_ref(values)
  updated_lengths = _sc_scatter(
      lengths, new_lengths, new_keys, new_values, k_ref, v_ref)
  return jax.freeze(k_ref), jax.freeze(v_ref), updated_lengths
